# Initial kernel scaffold; baseline (speedup 1.0000x reference)
import jax
import jax.numpy as jnp
from jax.experimental import pallas as pl


def kernel(predictedTensor, inputTensor, edge_index):
    raise NotImplementedError("write your pallas kernel here")


# speedup vs baseline: 8.9157x; 8.9157x over previous
import jax
import jax.numpy as jnp
from jax import lax
from jax.experimental import pallas as pl
from jax.experimental.pallas import tpu as pltpu
from jax.experimental.pallas import tpu_sc as plsc

N_NODES = 10000
N_EDGES = 320000
D = 128

NC = 2
NS = 16
NW = NC * NS
L = 16

N_PAD = 10240
CROWS = N_PAD // D
CHUNK = 128
CPT = 79
E_PAD = NW * CPT * CHUNK

ROWS_K2 = N_PAD // NS
ROWS_K3 = N_PAD // NW
RCH = 64


def _k1_body(pred_ref, inp_ref, out_ref):
    d = pred_ref[...] - inp_ref[...]
    out_ref[...] = jnp.concatenate(
        [d, jnp.zeros((N_PAD - N_NODES, D), jnp.float32)], axis=0)


def _k2_body(d_pad, rowi, coli, zrows, zcnt, out_part, out_cnt,
             acc_sh, cnt_sh, colv, rowv, rows, cntl, cbuf, cacc, sem):
    cid = lax.axis_index("c")
    sid = lax.axis_index("s")
    wid = cid * NS + sid

    pltpu.sync_copy(zrows.at[pl.ds(sid * ROWS_K2, ROWS_K2)],
                    acc_sh.at[pl.ds(sid * ROWS_K2, ROWS_K2)])
    pltpu.sync_copy(zcnt, cntl)
    plsc.subcore_barrier()

    ones = jnp.ones((L,), jnp.float32)

    def chunk(c, carry):
        pltpu.sync_copy(coli.at[wid, c], colv)
        pltpu.sync_copy(rowi.at[wid, c], rowv)
        pltpu.async_copy(d_pad.at[colv], rows, sem).wait()
        pltpu.sync_copy(rows, acc_sh.at[rowv], add=True)
        for j in range(CHUNK // L):
            plsc.addupdate_scatter(cntl, [rowv[pl.ds(j * L, L)]], ones)
        return carry

    lax.fori_loop(0, CPT, chunk, 0)

    pltpu.sync_copy(cntl, cnt_sh.at[sid])
    plsc.subcore_barrier()

    pltpu.sync_copy(acc_sh.at[pl.ds(sid * ROWS_K2, ROWS_K2)],
                    out_part.at[cid, pl.ds(sid * ROWS_K2, ROWS_K2)])

    pltpu.sync_copy(cnt_sh.at[0, pl.ds(sid * ROWS_K2, ROWS_K2)], cacc)
    for j in range(1, NS):
        pltpu.sync_copy(cnt_sh.at[j, pl.ds(sid * ROWS_K2, ROWS_K2)], cbuf)
        for t in range(ROWS_K2 // L):
            cacc[pl.ds(t * L, L)] += cbuf[pl.ds(t * L, L)]
    pltpu.sync_copy(cacc, out_cnt.at[cid, pl.ds(sid * ROWS_K2, ROWS_K2)])


def _k3_body(part, cnt, d_pad, out_p, s0b, s1b, db, c0b, c1b, accv):
    cid = lax.axis_index("c")
    sid = lax.axis_index("s")
    wid = cid * NS + sid
    base = wid * ROWS_K3

    accv[...] = jnp.zeros((L,), jnp.float32)
    for ch in range(ROWS_K3 // RCH):
        off = base + ch * RCH
        pltpu.sync_copy(part.at[0, pl.ds(off, RCH)], s0b)
        pltpu.sync_copy(part.at[1, pl.ds(off, RCH)], s1b)
        pltpu.sync_copy(d_pad.at[pl.ds(off, RCH)], db)
        pltpu.sync_copy(cnt.at[0, pl.ds(off, RCH)], c0b)
        pltpu.sync_copy(cnt.at[1, pl.ds(off, RCH)], c1b)

        def grp(g, vacc):
            cv = c0b[pl.ds(g * L, L)] + c1b[pl.ds(g * L, L)]
            gatev = jnp.where(cv > 0.0, 1.0, 0.0).astype(jnp.float32)
            invv = gatev / jnp.maximum(cv, 1.0)
            for r in range(L):
                row = g * L + r
                gate = jnp.full((L,), gatev[r], jnp.float32)
                inv = jnp.full((L,), invv[r], jnp.float32)
                for k in range(D // L):
                    s = s0b[row, pl.ds(k * L, L)] + s1b[row, pl.ds(k * L, L)]
                    v = gate * (db[row, pl.ds(k * L, L)] - s * inv)
                    vacc = vacc + v * v
            return vacc

        accv[...] = lax.fori_loop(0, RCH // L, grp, accv[...])

    pltpu.sync_copy(accv, out_p.at[cid, sid])


def _k4_body(p_ref, out_ref):
    out_ref[...] = (jnp.sum(p_ref[...]) * (1.0 / N_NODES)).reshape(1, 1)


def kernel(predictedTensor, inputTensor, edge_index):
    row = edge_index[0].astype(jnp.int32)
    col = edge_index[1].astype(jnp.int32)
    pad = jnp.full((E_PAD - N_EDGES,), N_NODES, jnp.int32)
    rowi = jnp.concatenate([row, pad]).reshape(NW, CPT, CHUNK)
    coli = jnp.concatenate([col, pad]).reshape(NW, CPT, CHUNK)
    zrows = jnp.zeros((N_PAD, D), jnp.float32)
    zcnt = jnp.zeros((N_PAD,), jnp.float32)

    d_pad = pl.pallas_call(
        _k1_body,
        out_shape=jax.ShapeDtypeStruct((N_PAD, D), jnp.float32),
    )(predictedTensor, inputTensor)

    mesh = plsc.VectorSubcoreMesh(core_axis_name="c", subcore_axis_name="s")

    k2 = pl.kernel(
        _k2_body,
        out_type=(
            jax.ShapeDtypeStruct((NC, N_PAD, D), jnp.float32),
            jax.ShapeDtypeStruct((NC, N_PAD), jnp.float32),
        ),
        mesh=mesh,
        scratch_types=[
            pltpu.VMEM_SHARED((N_PAD, D), jnp.float32),
            pltpu.VMEM_SHARED((NS, N_PAD), jnp.float32),
            pltpu.VMEM((CHUNK,), jnp.int32),
            pltpu.VMEM((CHUNK,), jnp.int32),
            pltpu.VMEM((CHUNK, D), jnp.float32),
            pltpu.VMEM((N_PAD,), jnp.float32),
            pltpu.VMEM((ROWS_K2,), jnp.float32),
            pltpu.VMEM((ROWS_K2,), jnp.float32),
            pltpu.SemaphoreType.DMA,
        ],
        compiler_params=pltpu.CompilerParams(needs_layout_passes=False),
    )
    part, cnts = k2(d_pad, rowi, coli, zrows, zcnt)

    k3 = pl.kernel(
        _k3_body,
        out_type=jax.ShapeDtypeStruct((NC, NS, L), jnp.float32),
        mesh=mesh,
        scratch_types=[
            pltpu.VMEM((RCH, D), jnp.float32),
            pltpu.VMEM((RCH, D), jnp.float32),
            pltpu.VMEM((RCH, D), jnp.float32),
            pltpu.VMEM((RCH,), jnp.float32),
            pltpu.VMEM((RCH,), jnp.float32),
            pltpu.VMEM((L,), jnp.float32),
        ],
    )
    partials = k3(part, cnts, d_pad)

    out = pl.pallas_call(
        _k4_body,
        out_shape=jax.ShapeDtypeStruct((1, 1), jnp.float32),
    )(partials)
    return out[0, 0]
